# Initial kernel scaffold; baseline (speedup 1.0000x reference)
import jax
import jax.numpy as jnp
from jax.experimental import pallas as pl


def kernel(x, edge_index, edge_weight, weight):
    raise NotImplementedError("write your pallas kernel here")


# speedup vs baseline: 4.4120x; 4.4120x over previous
import functools

import jax
import jax.numpy as jnp
from jax import lax
from jax.experimental import pallas as pl
from jax.experimental.pallas import tpu as pltpu
from jax.experimental.pallas import tpu_sc as plsc

N = 10000
E = 320000
D = 128

NUM_CORES = 2
NUM_SUBCORES = 16
NUM_WORKERS = NUM_CORES * NUM_SUBCORES
EDGES_PER_WORKER = E // NUM_WORKERS
CHUNK = 80
NUM_CHUNKS = EDGES_PER_WORKER // CHUNK
ROWS_PER_TILE = 624
COPY_ROWS = 16
LANES = 16


def _matmul(x, w):
    bm = 1000

    def body(x_ref, w_ref, o_ref):
        o_ref[...] = jnp.dot(x_ref[...], w_ref[...],
                             preferred_element_type=jnp.float32)

    return pl.pallas_call(
        body,
        grid=(N // bm,),
        in_specs=[
            pl.BlockSpec((bm, D), lambda i: (i, 0)),
            pl.BlockSpec((D, D), lambda i: (0, 0)),
        ],
        out_specs=pl.BlockSpec((bm, D), lambda i: (i, 0)),
        out_shape=jax.ShapeDtypeStruct((N, D), jnp.float32),
    )(x, w)


def _sum_partials(p):
    bm = 1000

    def body(p_ref, o_ref):
        o_ref[...] = p_ref[0] + p_ref[1]

    return pl.pallas_call(
        body,
        grid=(N // bm,),
        in_specs=[pl.BlockSpec((2, bm, D), lambda i: (0, i, 0))],
        out_specs=pl.BlockSpec((bm, D), lambda i: (i, 0)),
        out_shape=jax.ShapeDtypeStruct((N, D), jnp.float32),
    )(p)


def _make_sc_spmm():
    mesh = plsc.VectorSubcoreMesh(
        core_axis_name="c", subcore_axis_name="s",
        num_cores=NUM_CORES, num_subcores=NUM_SUBCORES)

    @functools.partial(
        pl.kernel,
        out_type=jax.ShapeDtypeStruct((NUM_CORES, N, D), jnp.float32),
        mesh=mesh,
        scratch_types=[
            pltpu.VMEM((CHUNK,), jnp.int32),
            pltpu.VMEM((CHUNK,), jnp.int32),
            pltpu.VMEM((CHUNK,), jnp.float32),
            pltpu.VMEM((CHUNK, D), jnp.float32),
            pltpu.VMEM((COPY_ROWS, D), jnp.float32),
            pltpu.VMEM_SHARED((N, D), jnp.float32),
            pltpu.SemaphoreType.DMA,
        ],
    )
    def spmm(z_hbm, cols_hbm, rows_hbm, ew_hbm, out_hbm,
             colv, rowv, ewv, gbuf, zbuf, accum, sem):
        c = lax.axis_index("c")
        s = lax.axis_index("s")

        zeros16 = jnp.zeros((LANES,), jnp.float32)
        for i in range(COPY_ROWS):
            for j in range(D // LANES):
                zbuf[i, pl.ds(j * LANES, LANES)] = zeros16

        row0 = s * ROWS_PER_TILE
        n_copy = (ROWS_PER_TILE + jnp.where(s == NUM_SUBCORES - 1, 16, 0)
                  ) // COPY_ROWS

        def zero_copy(t, _):
            pltpu.sync_copy(zbuf, accum.at[pl.ds(row0 + t * COPY_ROWS,
                                                 COPY_ROWS)])
            return 0

        lax.fori_loop(0, n_copy, zero_copy, 0)
        plsc.subcore_barrier()

        base = (c * NUM_SUBCORES + s) * EDGES_PER_WORKER

        def edge_body(k, _):
            off = base + k * CHUNK
            pltpu.sync_copy(cols_hbm.at[pl.ds(off, CHUNK)], colv)
            pltpu.sync_copy(ew_hbm.at[pl.ds(off, CHUNK)], ewv)
            pltpu.sync_copy(rows_hbm.at[pl.ds(off, CHUNK)], rowv)
            pltpu.async_copy(z_hbm.at[colv], gbuf, sem).wait()
            for g in range(CHUNK // LANES):
                ew_vec = ewv[pl.ds(g * LANES, LANES)]
                for e16 in range(LANES):
                    e = g * LANES + e16
                    wv = jnp.take(ew_vec,
                                  jnp.full((LANES,), e16, jnp.int32))
                    for j in range(D // LANES):
                        sl = pl.ds(j * LANES, LANES)
                        gbuf[e, sl] = gbuf[e, sl] * wv
            pltpu.sync_copy(gbuf, accum.at[rowv], add=True)
            return 0

        lax.fori_loop(0, NUM_CHUNKS, edge_body, 0)
        plsc.subcore_barrier()

        def out_copy(t, _):
            r = row0 + t * COPY_ROWS
            pltpu.sync_copy(accum.at[pl.ds(r, COPY_ROWS)], zbuf)
            pltpu.sync_copy(zbuf, out_hbm.at[c, pl.ds(r, COPY_ROWS)])
            return 0

        lax.fori_loop(0, n_copy, out_copy, 0)

    return spmm


_sc_spmm = _make_sc_spmm()


@jax.jit
def kernel(x, edge_index, edge_weight, weight):
    z = _matmul(x, weight)
    cols = edge_index[1]
    rows = edge_index[0]
    partials = _sc_spmm(z, cols, rows, edge_weight)
    return _sum_partials(partials)


# speedup vs baseline: 7.8820x; 1.7865x over previous
import functools

import jax
import jax.numpy as jnp
from jax import lax
from jax.experimental import pallas as pl
from jax.experimental.pallas import tpu as pltpu
from jax.experimental.pallas import tpu_sc as plsc

N = 10000
E = 320000
D = 128

NUM_CORES = 2
NUM_SUBCORES = 16
NUM_WORKERS = NUM_CORES * NUM_SUBCORES
EDGES_PER_WORKER = E // NUM_WORKERS
CHUNK = 80
NUM_CHUNKS = EDGES_PER_WORKER // CHUNK
NBUF = 4
ROWS_PER_TILE = 624
COPY_ROWS = 16
LANES = 16


def _matmul(x, w):
    bm = 1000

    def body(x_ref, w_ref, o_ref):
        o_ref[...] = jnp.dot(x_ref[...], w_ref[...],
                             preferred_element_type=jnp.float32)

    return pl.pallas_call(
        body,
        grid=(N // bm,),
        in_specs=[
            pl.BlockSpec((bm, D), lambda i: (i, 0)),
            pl.BlockSpec((D, D), lambda i: (0, 0)),
        ],
        out_specs=pl.BlockSpec((bm, D), lambda i: (i, 0)),
        out_shape=jax.ShapeDtypeStruct((N, D), jnp.float32),
    )(x, w)


def _sum_partials(p):
    bm = 1000

    def body(p_ref, o_ref):
        o_ref[...] = p_ref[0] + p_ref[1]

    return pl.pallas_call(
        body,
        grid=(N // bm,),
        in_specs=[pl.BlockSpec((2, bm, D), lambda i: (0, i, 0))],
        out_specs=pl.BlockSpec((bm, D), lambda i: (i, 0)),
        out_shape=jax.ShapeDtypeStruct((N, D), jnp.float32),
    )(p)


def _make_sc_spmm():
    mesh = plsc.VectorSubcoreMesh(
        core_axis_name="c", subcore_axis_name="s",
        num_cores=NUM_CORES, num_subcores=NUM_SUBCORES)

    @functools.partial(
        pl.kernel,
        out_type=jax.ShapeDtypeStruct((NUM_CORES, N, D), jnp.float32),
        mesh=mesh,
        scratch_types=[
            [pltpu.VMEM((CHUNK,), jnp.int32)] * NBUF,
            [pltpu.VMEM((CHUNK,), jnp.int32)] * NBUF,
            [pltpu.VMEM((CHUNK,), jnp.float32)] * NBUF,
            [pltpu.VMEM((CHUNK, D), jnp.float32)] * NBUF,
            pltpu.VMEM((COPY_ROWS, D), jnp.float32),
            pltpu.VMEM_SHARED((N, D), jnp.float32),
            [pltpu.SemaphoreType.DMA] * NBUF,
            [pltpu.SemaphoreType.DMA] * NBUF,
            [pltpu.SemaphoreType.DMA] * NBUF,
        ],
    )
    def spmm(z_hbm, cols_hbm, rows_hbm, ew_hbm, out_hbm,
             colv, rowv, ewv, gbuf, zbuf, accum, lsem, gsem, ssem):
        c = lax.axis_index("c")
        s = lax.axis_index("s")

        zeros16 = jnp.zeros((LANES,), jnp.float32)
        for i in range(COPY_ROWS):
            for j in range(D // LANES):
                zbuf[i, pl.ds(j * LANES, LANES)] = zeros16

        row0 = s * ROWS_PER_TILE
        n_copy = (ROWS_PER_TILE + jnp.where(s == NUM_SUBCORES - 1, 16, 0)
                  ) // COPY_ROWS

        def zero_copy(t, _):
            pltpu.sync_copy(zbuf, accum.at[pl.ds(row0 + t * COPY_ROWS,
                                                 COPY_ROWS)])
            return 0

        lax.fori_loop(0, n_copy, zero_copy, 0)
        plsc.subcore_barrier()

        base = (c * NUM_SUBCORES + s) * EDGES_PER_WORKER

        def load_idx(p, sl):
            off = base + p * CHUNK
            pltpu.async_copy(cols_hbm.at[pl.ds(off, CHUNK)], colv[sl],
                             lsem[sl])
            pltpu.async_copy(rows_hbm.at[pl.ds(off, CHUNK)], rowv[sl],
                             lsem[sl])
            pltpu.async_copy(ew_hbm.at[pl.ds(off, CHUNK)], ewv[sl], lsem[sl])

        def wait_idx(sl):
            pltpu.make_async_copy(cols_hbm.at[pl.ds(0, CHUNK)], colv[sl],
                                  lsem[sl]).wait()
            pltpu.make_async_copy(rows_hbm.at[pl.ds(0, CHUNK)], rowv[sl],
                                  lsem[sl]).wait()
            pltpu.make_async_copy(ew_hbm.at[pl.ds(0, CHUNK)], ewv[sl],
                                  lsem[sl]).wait()

        def start_gather(sl):
            pltpu.async_copy(z_hbm.at[colv[sl]], gbuf[sl], gsem[sl])

        def wait_gather(sl):
            pltpu.make_async_copy(z_hbm.at[colv[sl]], gbuf[sl],
                                  gsem[sl]).wait()

        def start_scatter(sl):
            pltpu.async_copy(gbuf[sl], accum.at[rowv[sl]], ssem[sl],
                             add=True)

        def wait_scatter(sl):
            pltpu.make_async_copy(gbuf[sl], accum.at[rowv[sl]],
                                  ssem[sl]).wait()

        def scale(sl):
            gb = gbuf[sl]
            ew = ewv[sl]

            def grp(g, _):
                ew_vec = ew[pl.ds(g * LANES, LANES)]
                for e16 in range(LANES):
                    wv = jnp.take(ew_vec, jnp.full((LANES,), e16, jnp.int32))
                    e = g * LANES + e16
                    for j in range(D // LANES):
                        sl2 = pl.ds(j * LANES, LANES)
                        gb[e, sl2] = gb[e, sl2] * wv
                return 0

            lax.fori_loop(0, CHUNK // LANES, grp, 0)

        for p in range(2):
            load_idx(p, p)
        wait_idx(0)
        start_gather(0)

        def steady(k2, _):
            for b in range(NBUF):
                a = k2 * NBUF + b
                wait_gather(b)
                scale(b)
                start_scatter(b)

                sl_l = (b + 2) % NBUF
                p_l = a + 2

                @pl.when(p_l < NUM_CHUNKS)
                def _():
                    @pl.when(p_l >= NBUF)
                    def _():
                        wait_scatter(sl_l)
                    load_idx(p_l, sl_l)

                sl_g = (b + 1) % NBUF
                p_g = a + 1

                @pl.when(p_g < NUM_CHUNKS)
                def _():
                    wait_idx(sl_g)
                    start_gather(sl_g)
            return 0

        lax.fori_loop(0, NUM_CHUNKS // NBUF, steady, 0)

        wait_gather(0)
        scale(0)
        start_scatter(0)

        for b in range(NBUF):
            wait_scatter(b)
        plsc.subcore_barrier()

        def out_copy(t, _):
            r = row0 + t * COPY_ROWS
            pltpu.sync_copy(accum.at[pl.ds(r, COPY_ROWS)], zbuf)
            pltpu.sync_copy(zbuf, out_hbm.at[c, pl.ds(r, COPY_ROWS)])
            return 0

        lax.fori_loop(0, n_copy, out_copy, 0)

    return spmm


_sc_spmm = _make_sc_spmm()


@jax.jit
def kernel(x, edge_index, edge_weight, weight):
    z = _matmul(x, weight)
    rows = edge_index[0]
    cols = edge_index[1]
    partials = _sc_spmm(z, cols, rows, edge_weight)
    return _sum_partials(partials)


# speedup vs baseline: 11.4164x; 1.4484x over previous
import functools

import jax
import jax.numpy as jnp
from jax import lax
from jax.experimental import pallas as pl
from jax.experimental.pallas import tpu as pltpu
from jax.experimental.pallas import tpu_sc as plsc

N = 10000
E = 320000
D = 128

NUM_CORES = 2
NUM_SUBCORES = 16
NUM_WORKERS = NUM_CORES * NUM_SUBCORES
EDGES_PER_WORKER = E // NUM_WORKERS
CHUNK = 80
NUM_CHUNKS = EDGES_PER_WORKER // CHUNK
NBUF = 4
LBUF = 8
ROWS_PER_TILE = 624
COPY_ROWS = 16
LANES = 16


def _matmul(x, w):
    bm = 1000

    def body(x_ref, w_ref, o_ref):
        o_ref[...] = jnp.dot(x_ref[...], w_ref[...],
                             preferred_element_type=jnp.float32)

    return pl.pallas_call(
        body,
        grid=(N // bm,),
        in_specs=[
            pl.BlockSpec((bm, D), lambda i: (i, 0)),
            pl.BlockSpec((D, D), lambda i: (0, 0)),
        ],
        out_specs=pl.BlockSpec((bm, D), lambda i: (i, 0)),
        out_shape=jax.ShapeDtypeStruct((N, D), jnp.float32),
    )(x, w)


def _sum_partials(p):
    bm = 1000

    def body(p_ref, o_ref):
        o_ref[...] = p_ref[0] + p_ref[1]

    return pl.pallas_call(
        body,
        grid=(N // bm,),
        in_specs=[pl.BlockSpec((2, bm, D), lambda i: (0, i, 0))],
        out_specs=pl.BlockSpec((bm, D), lambda i: (i, 0)),
        out_shape=jax.ShapeDtypeStruct((N, D), jnp.float32),
    )(p)


def _make_sc_spmm():
    mesh = plsc.VectorSubcoreMesh(
        core_axis_name="c", subcore_axis_name="s",
        num_cores=NUM_CORES, num_subcores=NUM_SUBCORES)

    @functools.partial(
        pl.kernel,
        out_type=jax.ShapeDtypeStruct((NUM_CORES, N, D), jnp.float32),
        mesh=mesh,
        scratch_types=[
            [pltpu.VMEM((CHUNK,), jnp.int32)] * LBUF,
            [pltpu.VMEM((CHUNK,), jnp.int32)] * LBUF,
            [pltpu.VMEM((CHUNK,), jnp.float32)] * LBUF,
            [pltpu.VMEM((CHUNK, D), jnp.float32)] * NBUF,
            pltpu.VMEM((COPY_ROWS, D), jnp.float32),
            pltpu.VMEM_SHARED((N, D), jnp.float32),
            [pltpu.SemaphoreType.DMA] * LBUF,
            [pltpu.SemaphoreType.DMA] * NBUF,
            [pltpu.SemaphoreType.DMA] * NBUF,
        ],
    )
    def spmm(z_hbm, cols_hbm, rows_hbm, ew_hbm, out_hbm,
             colv, rowv, ewv, gbuf, zbuf, accum, lsem, gsem, ssem):
        c = lax.axis_index("c")
        s = lax.axis_index("s")

        zeros16 = jnp.zeros((LANES,), jnp.float32)
        for i in range(COPY_ROWS):
            for j in range(D // LANES):
                zbuf[i, pl.ds(j * LANES, LANES)] = zeros16

        row0 = s * ROWS_PER_TILE
        n_copy = (ROWS_PER_TILE + jnp.where(s == NUM_SUBCORES - 1, 16, 0)
                  ) // COPY_ROWS

        def zero_copy(t, _):
            pltpu.sync_copy(zbuf, accum.at[pl.ds(row0 + t * COPY_ROWS,
                                                 COPY_ROWS)])
            return 0

        lax.fori_loop(0, n_copy, zero_copy, 0)
        plsc.subcore_barrier()

        base = (c * NUM_SUBCORES + s) * EDGES_PER_WORKER

        def load_idx(p, sl):
            off = base + p * CHUNK
            pltpu.async_copy(cols_hbm.at[pl.ds(off, CHUNK)], colv[sl],
                             lsem[sl])
            pltpu.async_copy(rows_hbm.at[pl.ds(off, CHUNK)], rowv[sl],
                             lsem[sl])
            pltpu.async_copy(ew_hbm.at[pl.ds(off, CHUNK)], ewv[sl], lsem[sl])

        def wait_idx(sl):
            pltpu.make_async_copy(cols_hbm.at[pl.ds(0, CHUNK)], colv[sl],
                                  lsem[sl]).wait()
            pltpu.make_async_copy(rows_hbm.at[pl.ds(0, CHUNK)], rowv[sl],
                                  lsem[sl]).wait()
            pltpu.make_async_copy(ew_hbm.at[pl.ds(0, CHUNK)], ewv[sl],
                                  lsem[sl]).wait()

        def start_gather(sl, lsl):
            pltpu.async_copy(z_hbm.at[colv[lsl]], gbuf[sl], gsem[sl])

        def wait_gather(sl):
            pltpu.make_async_copy(z_hbm.at[colv[0]], gbuf[sl],
                                  gsem[sl]).wait()

        def start_scatter(sl, lsl):
            pltpu.async_copy(gbuf[sl], accum.at[rowv[lsl]], ssem[sl],
                             add=True)

        def wait_scatter(sl):
            pltpu.make_async_copy(gbuf[sl], accum.at[rowv[0]],
                                  ssem[sl]).wait()

        def scale(sl, lsl):
            gb = gbuf[sl]
            ew = ewv[lsl]

            def grp(g, _):
                ew_vec = ew[pl.ds(g * LANES, LANES)]
                for e16 in range(LANES):
                    wv = jnp.take(ew_vec, jnp.full((LANES,), e16, jnp.int32))
                    e = g * LANES + e16
                    for j in range(D // LANES):
                        sl2 = pl.ds(j * LANES, LANES)
                        gb[e, sl2] = gb[e, sl2] * wv
                return 0

            lax.fori_loop(0, CHUNK // LANES, grp, 0)

        for p in range(4):
            load_idx(p, p)
        wait_idx(0)
        start_gather(0, 0)
        wait_idx(1)
        start_gather(1, 1)

        def chunk_step(a, b8, static_a=None):
            b = b8 % NBUF
            wait_gather(b)
            scale(b, b8)
            start_scatter(b, b8)

            p_l = a + 4
            lsl = (b8 + 4) % LBUF
            if static_a is None or static_a + 4 < NUM_CHUNKS:
                load_idx(p_l, lsl)

            p_g = a + 2
            gsl = (b + 2) % NBUF
            lg = (b8 + 2) % LBUF
            if static_a is None or static_a + 2 < NUM_CHUNKS:
                if static_a is None and b8 < 2:
                    @pl.when(p_g >= NBUF)
                    def _():
                        wait_scatter(gsl)
                elif static_a is None or static_a + 2 >= NBUF:
                    wait_scatter(gsl)
                wait_idx(lg)
                start_gather(gsl, lg)

        def steady(k8, _):
            for b8 in range(LBUF):
                chunk_step(k8 * LBUF + b8, b8)
            return 0

        n_steady = NUM_CHUNKS // LBUF
        lax.fori_loop(0, n_steady, steady, 0)

        for a in range(n_steady * LBUF, NUM_CHUNKS):
            chunk_step(a, a % LBUF, static_a=a)

        for b in range(NBUF):
            wait_scatter(b)
        plsc.subcore_barrier()

        def out_copy(t, _):
            r = row0 + t * COPY_ROWS
            pltpu.sync_copy(accum.at[pl.ds(r, COPY_ROWS)], zbuf)
            pltpu.sync_copy(zbuf, out_hbm.at[c, pl.ds(r, COPY_ROWS)])
            return 0

        lax.fori_loop(0, n_copy, out_copy, 0)

    return spmm


_sc_spmm = _make_sc_spmm()


@jax.jit
def kernel(x, edge_index, edge_weight, weight):
    z = _matmul(x, weight)
    rows = edge_index[0]
    cols = edge_index[1]
    partials = _sc_spmm(z, cols, rows, edge_weight)
    return _sum_partials(partials)


# speedup vs baseline: 11.8551x; 1.0384x over previous
import functools

import jax
import jax.numpy as jnp
from jax import lax
from jax.experimental import pallas as pl
from jax.experimental.pallas import tpu as pltpu
from jax.experimental.pallas import tpu_sc as plsc

N = 10000
E = 320000
D = 128

NUM_CORES = 2
NUM_SUBCORES = 16
NUM_WORKERS = NUM_CORES * NUM_SUBCORES
EDGES_PER_WORKER = E // NUM_WORKERS
CHUNK = 80
NUM_CHUNKS = EDGES_PER_WORKER // CHUNK
NBUF = 4
LBUF = 8
ROWS_PER_TILE = 624
COPY_ROWS = 16
LANES = 16


def _matmul(x, w):
    bm = 1000

    def body(x_ref, w_ref, o_ref):
        o_ref[...] = jnp.dot(x_ref[...], w_ref[...],
                             preferred_element_type=jnp.float32)

    return pl.pallas_call(
        body,
        grid=(N // bm,),
        in_specs=[
            pl.BlockSpec((bm, D), lambda i: (i, 0)),
            pl.BlockSpec((D, D), lambda i: (0, 0)),
        ],
        out_specs=pl.BlockSpec((bm, D), lambda i: (i, 0)),
        out_shape=jax.ShapeDtypeStruct((N, D), jnp.float32),
    )(x, w)


def _sum_partials(p):
    bm = 1000

    def body(p_ref, o_ref):
        o_ref[...] = p_ref[0] + p_ref[1]

    return pl.pallas_call(
        body,
        grid=(N // bm,),
        in_specs=[pl.BlockSpec((2, bm, D), lambda i: (0, i, 0))],
        out_specs=pl.BlockSpec((bm, D), lambda i: (i, 0)),
        out_shape=jax.ShapeDtypeStruct((N, D), jnp.float32),
    )(p)


def _make_sc_spmm():
    mesh = plsc.VectorSubcoreMesh(
        core_axis_name="c", subcore_axis_name="s",
        num_cores=NUM_CORES, num_subcores=NUM_SUBCORES)

    @functools.partial(
        pl.kernel,
        out_type=jax.ShapeDtypeStruct((NUM_CORES, N, D), jnp.float32),
        mesh=mesh,
        scratch_types=[
            [pltpu.VMEM((CHUNK,), jnp.int32)] * LBUF,
            [pltpu.VMEM((CHUNK,), jnp.int32)] * LBUF,
            [pltpu.VMEM((CHUNK,), jnp.float32)] * LBUF,
            [pltpu.VMEM((CHUNK, D), jnp.float32)] * NBUF,
            pltpu.VMEM((COPY_ROWS, D), jnp.float32),
            pltpu.VMEM_SHARED((N, D), jnp.float32),
            [pltpu.SemaphoreType.DMA] * LBUF,
            [pltpu.SemaphoreType.DMA] * NBUF,
            [pltpu.SemaphoreType.DMA] * NBUF,
        ],
    )
    def spmm(z_hbm, cols_hbm, rows_hbm, ew_hbm, out_hbm,
             colv, rowv, ewv, gbuf, zbuf, accum, lsem, gsem, ssem):
        c = lax.axis_index("c")
        s = lax.axis_index("s")

        zeros16 = jnp.zeros((LANES,), jnp.float32)
        for i in range(COPY_ROWS):
            for j in range(D // LANES):
                zbuf[i, pl.ds(j * LANES, LANES)] = zeros16

        row0 = s * ROWS_PER_TILE
        n_copy = (ROWS_PER_TILE + jnp.where(s == NUM_SUBCORES - 1, 16, 0)
                  ) // COPY_ROWS

        def zero_copy(t, _):
            pltpu.sync_copy(zbuf, accum.at[pl.ds(row0 + t * COPY_ROWS,
                                                 COPY_ROWS)])
            return 0

        lax.fori_loop(0, n_copy, zero_copy, 0)
        plsc.subcore_barrier()

        base = (c * NUM_SUBCORES + s) * EDGES_PER_WORKER

        def load_idx(p, sl):
            off = base + p * CHUNK
            pltpu.async_copy(cols_hbm.at[pl.ds(off, CHUNK)], colv[sl],
                             lsem[sl])
            pltpu.async_copy(rows_hbm.at[pl.ds(off, CHUNK)], rowv[sl],
                             lsem[sl])
            pltpu.async_copy(ew_hbm.at[pl.ds(off, CHUNK)], ewv[sl], lsem[sl])

        def wait_idx(sl):
            pltpu.make_async_copy(cols_hbm.at[pl.ds(0, CHUNK)], colv[sl],
                                  lsem[sl]).wait()
            pltpu.make_async_copy(rows_hbm.at[pl.ds(0, CHUNK)], rowv[sl],
                                  lsem[sl]).wait()
            pltpu.make_async_copy(ew_hbm.at[pl.ds(0, CHUNK)], ewv[sl],
                                  lsem[sl]).wait()

        def start_gather(sl, lsl):
            pltpu.async_copy(z_hbm.at[colv[lsl]], gbuf[sl], gsem[sl])

        def wait_gather(sl):
            pltpu.make_async_copy(z_hbm.at[colv[0]], gbuf[sl],
                                  gsem[sl]).wait()

        def start_scatter(sl, lsl):
            pltpu.async_copy(gbuf[sl], accum.at[rowv[lsl]], ssem[sl],
                             add=True)

        def wait_scatter(sl):
            pltpu.make_async_copy(gbuf[sl], accum.at[rowv[0]],
                                  ssem[sl]).wait()

        def scale(sl, lsl):
            gb = gbuf[sl]
            ew = ewv[lsl]

            def grp(g, _):
                ew_vec = ew[pl.ds(g * LANES, LANES)]
                for e16 in range(LANES):
                    wv = jnp.take(ew_vec, jnp.full((LANES,), e16, jnp.int32))
                    e = g * LANES + e16
                    for j in range(D // LANES):
                        sl2 = pl.ds(j * LANES, LANES)
                        gb[e, sl2] = gb[e, sl2] * wv
                return 0

            lax.fori_loop(0, CHUNK // LANES, grp, 0)

        for p in range(4):
            load_idx(p, p)
        wait_idx(0)
        start_gather(0, 0)
        wait_idx(1)
        start_gather(1, 1)

        def chunk_step(a, b8, static_a=None):
            b = b8 % NBUF

            p_g = a + 2
            gsl = (b + 2) % NBUF
            lg = (b8 + 2) % LBUF
            if static_a is None or static_a + 2 < NUM_CHUNKS:
                if static_a is None and b8 < 2:
                    @pl.when(p_g >= NBUF)
                    def _():
                        wait_scatter(gsl)
                elif static_a is None or static_a + 2 >= NBUF:
                    wait_scatter(gsl)
                wait_idx(lg)
                start_gather(gsl, lg)

            p_l = a + 4
            lsl = (b8 + 4) % LBUF
            if static_a is None or static_a + 4 < NUM_CHUNKS:
                load_idx(p_l, lsl)

            wait_gather(b)
            scale(b, b8)
            start_scatter(b, b8)

        def steady(k8, _):
            for b8 in range(LBUF):
                chunk_step(k8 * LBUF + b8, b8)
            return 0

        n_steady = NUM_CHUNKS // LBUF
        lax.fori_loop(0, n_steady, steady, 0)

        for a in range(n_steady * LBUF, NUM_CHUNKS):
            chunk_step(a, a % LBUF, static_a=a)

        for b in range(NBUF):
            wait_scatter(b)
        plsc.subcore_barrier()

        def out_copy(t, _):
            r = row0 + t * COPY_ROWS
            pltpu.sync_copy(accum.at[pl.ds(r, COPY_ROWS)], zbuf)
            pltpu.sync_copy(zbuf, out_hbm.at[c, pl.ds(r, COPY_ROWS)])
            return 0

        lax.fori_loop(0, n_copy, out_copy, 0)

    return spmm


_sc_spmm = _make_sc_spmm()


@jax.jit
def kernel(x, edge_index, edge_weight, weight):
    z = _matmul(x, weight)
    rows = edge_index[0]
    cols = edge_index[1]
    partials = _sc_spmm(z, cols, rows, edge_weight)
    return _sum_partials(partials)
